# baseline (speedup 1.0000x reference)
import functools

import jax
import jax.numpy as jnp
from jax import lax
from jax.experimental import pallas as pl
from jax.experimental.pallas import tpu as pltpu
from jax.experimental.pallas import tpu_sc as plsc

_E = 320000
_D = 128
_NC = 2
_NS = 16
_L = 16
_NW = _NC * _NS
_EPW = _E // _NW
_C = 400
_CHUNKS = _EPW // _C
_PAIRS = (_CHUNKS - 1) // 2


def _vsqrt(x):
    xi = lax.bitcast_convert_type(x, jnp.int32)
    yi = (xi >> 1) + jnp.int32(0x1FBD1DF5)
    y = lax.bitcast_convert_type(yi, jnp.float32)
    y = 0.5 * (y + x / y)
    y = 0.5 * (y + x / y)
    return y


_mesh = plsc.VectorSubcoreMesh(core_axis_name="c", subcore_axis_name="s")

_scratch = [
    pltpu.VMEM((_C,), jnp.int32),
    pltpu.VMEM((_C,), jnp.int32),
    pltpu.VMEM((_C,), jnp.int32),
    pltpu.VMEM((_C,), jnp.int32),
    pltpu.VMEM((_C, _D), jnp.float32),
    pltpu.VMEM((_C, _D), jnp.float32),
    pltpu.VMEM((_L * _L,), jnp.float32),
    pltpu.VMEM((_L,), jnp.float32),
    pltpu.SemaphoreType.DMA,
    pltpu.SemaphoreType.DMA,
    pltpu.SemaphoreType.DMA,
    pltpu.SemaphoreType.DMA,
]


@functools.partial(
    pl.kernel,
    out_type=jax.ShapeDtypeStruct((_NW, _L), jnp.float32),
    mesh=_mesh,
    compiler_params=pltpu.CompilerParams(needs_layout_passes=False),
    scratch_types=_scratch,
)
def _sc_loss(feat_hbm, fneg_hbm, eidx_hbm, out_hbm,
             sidxA, didxA, sidxB, didxB, bufA, bufB, tmp, tot_v,
             semA, semB, semIA, semIB):
    wid = lax.axis_index("s") * _NC + lax.axis_index("c")
    wbase = wid * _EPW
    lane = lax.iota(jnp.int32, _L)

    def issue_idx(n, sidx, didx, sem):
        base = wbase + n * _C
        pltpu.async_copy(eidx_hbm.at[pl.ds(base, _C)], sidx, sem)
        pltpu.async_copy(eidx_hbm.at[pl.ds(_E + base, _C)], didx, sem)

    def wait_idx(sidx, didx, sem):
        pltpu.make_async_copy(eidx_hbm.at[pl.ds(0, _C)], sidx, sem).wait()
        pltpu.make_async_copy(eidx_hbm.at[pl.ds(0, _C)], didx, sem).wait()

    def issue_g1(sidx, buf, sem):
        pltpu.async_copy(feat_hbm.at[sidx], buf, sem)

    def wait_g1(sidx, buf, sem):
        pltpu.make_async_copy(feat_hbm.at[sidx], buf, sem).wait()

    def issue_g2(didx, buf, sem):
        pltpu.async_copy(fneg_hbm.at[didx], buf, sem, add=True)

    def wait_g2(didx, buf, sem):
        pltpu.make_async_copy(fneg_hbm.at[didx], buf, sem).wait()

    def compute(buf, total):
        def group_body(i, tot):
            base = i * _L
            for j in range(_L):
                row = base + j
                acc = None
                for b in range(_D // _L):
                    df = buf[row, pl.ds(b * _L, _L)]
                    sq = df * df
                    acc = sq if acc is None else acc + sq
                plsc.store_scatter(tmp, [lane * _L + j], acc)
            sq16 = tmp[pl.ds(0, _L)]
            for l in range(1, _L):
                sq16 = sq16 + tmp[pl.ds(l * _L, _L)]
            return tot + _vsqrt(sq16)

        return lax.fori_loop(0, _C // _L, group_body, total)

    issue_idx(0, sidxA, didxA, semIA)
    issue_idx(1, sidxB, didxB, semIB)
    wait_idx(sidxA, didxA, semIA)
    issue_g1(sidxA, bufA, semA)
    wait_g1(sidxA, bufA, semA)
    issue_g2(didxA, bufA, semA)

    def pair_body(p, total):
        wait_idx(sidxB, didxB, semIB)
        issue_g1(sidxB, bufB, semB)
        wait_g2(didxA, bufA, semA)
        issue_idx(2 * p + 2, sidxA, didxA, semIA)
        total = compute(bufA, total)
        wait_g1(sidxB, bufB, semB)
        issue_g2(didxB, bufB, semB)
        wait_idx(sidxA, didxA, semIA)
        issue_g1(sidxA, bufA, semA)
        wait_g2(didxB, bufB, semB)

        @pl.when(p < _PAIRS - 1)
        def _():
            issue_idx(2 * p + 3, sidxB, didxB, semIB)

        total = compute(bufB, total)
        wait_g1(sidxA, bufA, semA)
        issue_g2(didxA, bufA, semA)
        return total

    total = lax.fori_loop(0, _PAIRS, pair_body, jnp.zeros((_L,), jnp.float32))

    wait_g2(didxA, bufA, semA)
    total = compute(bufA, total)

    tot_v[...] = total
    pltpu.sync_copy(tot_v, out_hbm.at[wid])


def kernel(features, edge_index):
    partials = _sc_loss(features, -features, edge_index.reshape(-1))
    return jnp.sum(partials) * (1.0 / _E)


# speedup vs baseline: 1.5973x; 1.5973x over previous
import functools

import jax
import jax.numpy as jnp
from jax import lax
from jax.experimental import pallas as pl
from jax.experimental.pallas import tpu as pltpu
from jax.experimental.pallas import tpu_sc as plsc

_E = 320000
_D = 128
_NC = 2
_NS = 16
_L = 16
_NW = _NC * _NS
_EPW = _E // _NW
_C = 80
_CHUNKS = _EPW // _C
_R = 5


def _vsqrt(x):
    xi = lax.bitcast_convert_type(x, jnp.int32)
    yi = (xi >> 1) + jnp.int32(0x1FBD1DF5)
    y = lax.bitcast_convert_type(yi, jnp.float32)
    y = 0.5 * (y + x / y)
    y = 0.5 * (y + x / y)
    return y


_mesh = plsc.VectorSubcoreMesh(core_axis_name="c", subcore_axis_name="s")

_scratch = (
    [
        pltpu.VMEM((_EPW,), jnp.int32),
        pltpu.VMEM((_EPW,), jnp.int32),
    ]
    + [pltpu.VMEM((_C, _D), jnp.float32) for _ in range(_R)]
    + [
        pltpu.VMEM((_L * _L,), jnp.float32),
        pltpu.VMEM((_L,), jnp.float32),
    ]
    + [pltpu.SemaphoreType.DMA for _ in range(_R)]
    + [pltpu.SemaphoreType.DMA]
)


@functools.partial(
    pl.kernel,
    out_type=jax.ShapeDtypeStruct((_NW, _L), jnp.float32),
    mesh=_mesh,
    compiler_params=pltpu.CompilerParams(needs_layout_passes=False),
    scratch_types=_scratch,
)
def _sc_loss(feat_hbm, fneg_hbm, eidx_hbm, out_hbm, *scr):
    sidx_all, didx_all = scr[0], scr[1]
    dbuf = scr[2:2 + _R]
    tmp = scr[2 + _R]
    tot_v = scr[3 + _R]
    semG = scr[4 + _R:4 + 2 * _R]
    semI = scr[4 + 2 * _R]

    wid = lax.axis_index("s") * _NC + lax.axis_index("c")
    wbase = wid * _EPW
    lane = lax.iota(jnp.int32, _L)

    def issue_g1(n, k):
        pltpu.async_copy(
            feat_hbm.at[sidx_all.at[pl.ds(n * _C, _C)]], dbuf[k], semG[k])

    def wait_g1(k):
        pltpu.make_async_copy(
            feat_hbm.at[sidx_all.at[pl.ds(0, _C)]], dbuf[k], semG[k]).wait()

    def issue_g2(n, k):
        pltpu.async_copy(
            fneg_hbm.at[didx_all.at[pl.ds(n * _C, _C)]], dbuf[k], semG[k],
            add=True)

    def wait_g2(k):
        pltpu.make_async_copy(
            fneg_hbm.at[didx_all.at[pl.ds(0, _C)]], dbuf[k], semG[k]).wait()

    def compute(k, total):
        rows = dbuf[k]

        def group_body(i, tot):
            base = i * _L
            for j in range(_L):
                row = base + j
                acc = None
                for b in range(_D // _L):
                    df = rows[row, pl.ds(b * _L, _L)]
                    sq = df * df
                    acc = sq if acc is None else acc + sq
                plsc.store_scatter(tmp, [lane * _L + j], acc)
            sq16 = tmp[pl.ds(0, _L)]
            for l in range(1, _L):
                sq16 = sq16 + tmp[pl.ds(l * _L, _L)]
            return tot + _vsqrt(sq16)

        return lax.fori_loop(0, _C // _L, group_body, total)

    pltpu.async_copy(eidx_hbm.at[pl.ds(wbase, _EPW)], sidx_all, semI)
    pltpu.async_copy(eidx_hbm.at[pl.ds(_E + wbase, _EPW)], didx_all, semI)
    pltpu.make_async_copy(eidx_hbm.at[pl.ds(0, _EPW)], sidx_all, semI).wait()
    pltpu.make_async_copy(eidx_hbm.at[pl.ds(0, _EPW)], didx_all, semI).wait()
    for i in range(4):
        issue_g1(i, i)
    wait_g1(0)
    issue_g2(0, 0)
    wait_g1(1)
    issue_g2(1, 1)

    def ring_body(p, total):
        n0 = p * _R
        for k in range(_R):
            n = n0 + k

            @pl.when(n + 4 < _CHUNKS)
            def _():
                issue_g1(n + 4, (k + 4) % _R)

            @pl.when(n + 2 < _CHUNKS)
            def _():
                wait_g1((k + 2) % _R)
                issue_g2(n + 2, (k + 2) % _R)

            wait_g2(k)
            total = compute(k, total)
        return total

    total = lax.fori_loop(0, _CHUNKS // _R, ring_body,
                          jnp.zeros((_L,), jnp.float32))

    tot_v[...] = total
    pltpu.sync_copy(tot_v, out_hbm.at[wid])


def kernel(features, edge_index):
    partials = _sc_loss(features, -features, edge_index.reshape(-1))
    return jnp.sum(partials) * (1.0 / _E)
